# baseline (speedup 1.0000x reference)
import functools

import jax
import jax.numpy as jnp
from jax import lax
from jax.experimental import pallas as pl
from jax.experimental.pallas import tpu as pltpu
from jax.experimental.pallas import tpu_sc as plsc

_N = 10000
_NP = 10112
_D = 128
_H = 128
_C = 8
_E = 320000
_NC = 2
_NS = 16
_NW = _NC * _NS
_CH = 128
_EP = 327680
_RPT = _NP // _NS
_BLK = _NP // 16
_GRID = _NP // _BLK

_IW = 80
_NR = _EP // _IW
_RPW = _NR // _NW
_NSL = 4
_QR = _RPW // _NSL
_NBUF = 4

_DW = 128
_DNR = _EP // _DW
_DRPW = _DNR // _NW


def _mesh():
    return plsc.VectorSubcoreMesh(
        core_axis_name="c", subcore_axis_name="s",
        num_cores=_NC, num_subcores=_NS)


def _deg_body(dst_hbm, deg_out, dst_sl, buf, deg_acc):
    c = lax.axis_index("c")
    s = lax.axis_index("s")
    w = c * _NS + s

    def fillz(i, carry):
        for k in range(_H // 16):
            buf[i, pl.ds(k * 16, 16)] = jnp.zeros((16,), jnp.float32)
        return carry
    lax.fori_loop(0, _CH, fillz, 0)
    off = 0
    for sz in [_CH] * (_RPT // _CH) + [_RPT % _CH]:
        pltpu.sync_copy(buf.at[pl.ds(0, sz)],
                        deg_acc.at[pl.ds(s * _RPT + off, sz)])
        off += sz

    def fillo(i, carry):
        for k in range(_H // 16):
            buf[i, pl.ds(k * 16, 16)] = jnp.ones((16,), jnp.float32)
        return carry
    lax.fori_loop(0, _DW, fillo, 0)
    plsc.subcore_barrier()

    pltpu.sync_copy(dst_hbm.at[pl.ds(w * _DRPW, _DRPW)], dst_sl)

    def chunk(j, carry):
        pltpu.sync_copy(buf, deg_acc.at[dst_sl.at[j]], add=True)
        return carry
    lax.fori_loop(0, _DRPW, chunk, 0)
    plsc.subcore_barrier()

    pltpu.sync_copy(deg_acc.at[pl.ds(s * _RPT, _RPT)],
                    deg_out.at[c, pl.ds(s * _RPT, _RPT)])


_deg_kernel = pl.kernel(
    _deg_body,
    out_type=jax.ShapeDtypeStruct((_NC, _NP, _H), jnp.float32),
    mesh=_mesh(),
    scratch_types=[
        pltpu.VMEM((_DRPW, _DW), jnp.int32),
        pltpu.VMEM((_CH, _H), jnp.float32),
        pltpu.VMEM_SHARED((_NP, _H), jnp.float32),
    ],
)


def _msg_body(xs_hbm, src_hbm, dst_hbm, out_hbm,
              src_sl, dst_sl, bufs, acc, *sems):
    c = lax.axis_index("c")
    s = lax.axis_index("s")
    w = c * _NS + s

    def fill(i, carry):
        for k in range(_D // 16):
            bufs[i, pl.ds(k * 16, 16)] = jnp.zeros((16,), jnp.float32)
        return carry
    lax.fori_loop(0, _NBUF * _IW, fill, 0)

    off = 0
    for sz in [_CH] * (_RPT // _CH) + [_RPT % _CH]:
        pltpu.sync_copy(bufs.at[pl.ds(0, sz)],
                        acc.at[pl.ds(s * _RPT + off, sz)])
        off += sz
    plsc.subcore_barrier()

    for h in range(_NSL):
        pltpu.sync_copy(src_hbm.at[pl.ds(w * _RPW + h * _QR, _QR)], src_sl)
        pltpu.sync_copy(dst_hbm.at[pl.ds(w * _RPW + h * _QR, _QR)], dst_sl)
        for b in range(_NBUF):
            pltpu.async_copy(xs_hbm.at[src_sl.at[b]],
                             bufs.at[pl.ds(b * _IW, _IW)], sems[b])

        def body(kk, carry):
            j0 = _NBUF * kk
            for b in range(_NBUF):
                j = j0 + b
                bref = bufs.at[pl.ds(b * _IW, _IW)]
                pltpu.make_async_copy(xs_hbm.at[src_sl.at[j]], bref,
                                      sems[b]).wait()
                pltpu.sync_copy(bref, acc.at[dst_sl.at[j]], add=True)

                @pl.when(kk < _QR // _NBUF - 1)
                def _():
                    pltpu.async_copy(xs_hbm.at[src_sl.at[j + _NBUF]], bref,
                                     sems[b])
            return carry
        lax.fori_loop(0, _QR // _NBUF, body, 0)
    plsc.subcore_barrier()

    pltpu.sync_copy(acc.at[pl.ds(s * _RPT, _RPT)],
                    out_hbm.at[c, pl.ds(s * _RPT, _RPT)])


_msg_kernel = pl.kernel(
    _msg_body,
    out_type=jax.ShapeDtypeStruct((_NC, _NP, _H), jnp.float32),
    mesh=_mesh(),
    scratch_types=[
        pltpu.VMEM((_QR, _IW), jnp.int32),
        pltpu.VMEM((_QR, _IW), jnp.int32),
        pltpu.VMEM((_NBUF * _IW, _D), jnp.float32),
        pltpu.VMEM_SHARED((_NP, _H), jnp.float32),
    ] + [pltpu.SemaphoreType.DMA] * _NBUF,
)


def _tcA_body(x_ref, w_ref, da_ref, db_ref, xs_ref, dv_ref):
    xw = jnp.dot(x_ref[...], w_ref[...], preferred_element_type=jnp.float32)
    deg = da_ref[:, 0:1] + db_ref[:, 0:1] + 1.0
    dinv = lax.rsqrt(jnp.maximum(deg, 1e-12))
    xs_ref[...] = xw * dinv
    dv_ref[...] = jnp.broadcast_to(dinv, (_BLK, _H))


def _ln_relu(agg, g_ref, be_ref):
    mu = jnp.mean(agg, axis=-1, keepdims=True)
    var = jnp.mean((agg - mu) ** 2, axis=-1, keepdims=True)
    ln = (agg - mu) * lax.rsqrt(var + 1e-5) * g_ref[...] + be_ref[...]
    return jnp.maximum(ln, 0.0)


def _tcB_body(pa_ref, pb_ref, xs_ref, dv_ref, b_ref, g_ref, be_ref, w2_ref,
              out_ref):
    agg = dv_ref[...] * (pa_ref[...] + pb_ref[...] + xs_ref[...]) + b_ref[...]
    h = _ln_relu(agg, g_ref, be_ref)
    xw = jnp.dot(h, w2_ref[...], preferred_element_type=jnp.float32)
    out_ref[...] = xw * dv_ref[:, 0:1]


def _tcC_body(pa_ref, pb_ref, xs_ref, dv_ref, b_ref, g_ref, be_ref,
              wh1_ref, bh1_ref, wh2_ref, bh2_ref, out_ref):
    agg = dv_ref[...] * (pa_ref[...] + pb_ref[...] + xs_ref[...]) + b_ref[...]
    h = _ln_relu(agg, g_ref, be_ref)
    h = jnp.maximum(
        jnp.dot(h, wh1_ref[...], preferred_element_type=jnp.float32)
        + bh1_ref[...], 0.0)
    out_ref[...] = (jnp.dot(h, wh2_ref[...], preferred_element_type=jnp.float32)
                    + bh2_ref[...])


def _row_spec():
    return pl.BlockSpec((_BLK, _H), lambda i: (i, 0))


def _full_spec(r, c):
    return pl.BlockSpec((r, c), lambda i: (0, 0))


_tcA = pl.pallas_call(
    _tcA_body,
    grid=(_GRID,),
    in_specs=[_row_spec(), _full_spec(_D, _H),
              _row_spec(), _row_spec()],
    out_specs=[_row_spec(), _row_spec()],
    out_shape=[jax.ShapeDtypeStruct((_NP, _H), jnp.float32),
               jax.ShapeDtypeStruct((_NP, _H), jnp.float32)],
)

_tcB = pl.pallas_call(
    _tcB_body,
    grid=(_GRID,),
    in_specs=[_row_spec(), _row_spec(), _row_spec(), _row_spec(),
              _full_spec(1, _H), _full_spec(1, _H), _full_spec(1, _H),
              _full_spec(_H, _H)],
    out_specs=_row_spec(),
    out_shape=jax.ShapeDtypeStruct((_NP, _H), jnp.float32),
)

_tcC = pl.pallas_call(
    _tcC_body,
    grid=(_GRID,),
    in_specs=[_row_spec(), _row_spec(), _row_spec(), _row_spec(),
              _full_spec(1, _H), _full_spec(1, _H), _full_spec(1, _H),
              _full_spec(_H, _H), _full_spec(1, _H), _full_spec(_H, _H),
              _full_spec(1, _H)],
    out_specs=_row_spec(),
    out_shape=jax.ShapeDtypeStruct((_NP, _H), jnp.float32),
)


def kernel(x, edge_index, W1, b1, g1, be1, W2, b2, g2, be2, Wh1, bh1, Wh2, bh2):
    f32 = jnp.float32
    x_pad = jnp.pad(x.astype(f32), ((0, _NP - _N), (0, 0)))
    pad = jnp.full((_EP - _E,), _N, jnp.int32)
    src_flat = jnp.concatenate([edge_index[0].astype(jnp.int32), pad])
    dst_flat = jnp.concatenate([edge_index[1].astype(jnp.int32), pad])
    src = src_flat.reshape(_NR, _IW)
    dst = dst_flat.reshape(_NR, _IW)
    dstd = dst_flat.reshape(_DNR, _DW)

    deg2 = _deg_kernel(dstd)
    xs1, dinvb = _tcA(x_pad, W1, deg2[0], deg2[1])

    p1 = _msg_kernel(xs1, src, dst)
    xs2 = _tcB(p1[0], p1[1], xs1, dinvb,
               b1.reshape(1, -1), g1.reshape(1, -1), be1.reshape(1, -1), W2)

    p2 = _msg_kernel(xs2, src, dst)
    wh2p = jnp.pad(Wh2, ((0, 0), (0, _H - _C)))
    bh2p = jnp.pad(bh2, (0, _H - _C)).reshape(1, -1)
    outp = _tcC(p2[0], p2[1], xs2, dinvb,
                b2.reshape(1, -1), g2.reshape(1, -1), be2.reshape(1, -1),
                Wh1, bh1.reshape(1, -1), wh2p, bh2p)
    return outp[:_N, :_C]


# speedup vs baseline: 1.1077x; 1.1077x over previous
import functools

import jax
import jax.numpy as jnp
from jax import lax
from jax.experimental import pallas as pl
from jax.experimental.pallas import tpu as pltpu
from jax.experimental.pallas import tpu_sc as plsc

_N = 10000
_NP = 10240
_D = 128
_H = 128
_C = 8
_E = 320000
_NC = 2
_NS = 16
_NW = _NC * _NS
_CH = 128
_T = 80
_EP = _NW * _T * _CH
_RPT = _NP // _NS
_BLK = 1024
_GRID = _NP // _BLK

_IW = 64
_NR = _EP // _IW
_RPW = _NR // _NW
_NSL = 4
_QR = _RPW // _NSL
_NBUF = 4


def _mesh():
    return plsc.VectorSubcoreMesh(
        core_axis_name="c", subcore_axis_name="s",
        num_cores=_NC, num_subcores=_NS)


def _deg_body(dst_hbm, deg_out, dst_sl, buf, deg_acc):
    c = lax.axis_index("c")
    s = lax.axis_index("s")
    w = c * _NS + s

    def fillz(i, carry):
        for k in range(_H // 16):
            buf[i, pl.ds(k * 16, 16)] = jnp.zeros((16,), jnp.float32)
        return carry
    lax.fori_loop(0, _CH, fillz, 0)
    for k in range(_RPT // _CH):
        pltpu.sync_copy(buf.at[pl.ds(0, _CH)],
                        deg_acc.at[pl.ds(s * _RPT + k * _CH, _CH)])

    def fillo(i, carry):
        for k in range(_H // 16):
            buf[i, pl.ds(k * 16, 16)] = jnp.ones((16,), jnp.float32)
        return carry
    lax.fori_loop(0, _IW, fillo, 0)
    plsc.subcore_barrier()

    pltpu.sync_copy(dst_hbm.at[pl.ds(w * _RPW, _RPW)], dst_sl)

    def chunk(j, carry):
        pltpu.sync_copy(buf.at[pl.ds(0, _IW)], deg_acc.at[dst_sl.at[j]],
                        add=True)
        return carry
    lax.fori_loop(0, _RPW, chunk, 0)
    plsc.subcore_barrier()

    pltpu.sync_copy(deg_acc.at[pl.ds(s * _RPT, _RPT)],
                    deg_out.at[c, pl.ds(s * _RPT, _RPT)])


_deg_kernel = pl.kernel(
    _deg_body,
    out_type=jax.ShapeDtypeStruct((_NC, _NP, _H), jnp.float32),
    mesh=_mesh(),
    scratch_types=[
        pltpu.VMEM((_RPW, _IW), jnp.int32),
        pltpu.VMEM((_CH, _H), jnp.float32),
        pltpu.VMEM_SHARED((_NP, _H), jnp.float32),
    ],
)


def _msg_body(xs_hbm, src_hbm, dst_hbm, out_hbm,
              src_sl, dst_sl, bufs, acc, *sems):
    c = lax.axis_index("c")
    s = lax.axis_index("s")
    w = c * _NS + s

    def fill(i, carry):
        for k in range(_D // 16):
            bufs[i, pl.ds(k * 16, 16)] = jnp.zeros((16,), jnp.float32)
        return carry
    lax.fori_loop(0, _NBUF * _IW, fill, 0)

    for k in range(_RPT // _CH):
        pltpu.sync_copy(bufs.at[pl.ds(0, _CH)],
                        acc.at[pl.ds(s * _RPT + k * _CH, _CH)])
    plsc.subcore_barrier()

    for h in range(_NSL):
        pltpu.sync_copy(src_hbm.at[pl.ds(w * _RPW + h * _QR, _QR)], src_sl)
        pltpu.sync_copy(dst_hbm.at[pl.ds(w * _RPW + h * _QR, _QR)], dst_sl)
        for b in range(_NBUF):
            pltpu.async_copy(xs_hbm.at[src_sl.at[b]],
                             bufs.at[pl.ds(b * _IW, _IW)], sems[b])

        def body(kk, carry):
            j0 = _NBUF * kk
            for b in range(_NBUF):
                j = j0 + b
                bref = bufs.at[pl.ds(b * _IW, _IW)]
                pltpu.make_async_copy(xs_hbm.at[src_sl.at[j]], bref,
                                      sems[b]).wait()
                pltpu.sync_copy(bref, acc.at[dst_sl.at[j]], add=True)

                @pl.when(kk < _QR // _NBUF - 1)
                def _():
                    pltpu.async_copy(xs_hbm.at[src_sl.at[j + _NBUF]], bref,
                                     sems[b])
            return carry
        lax.fori_loop(0, _QR // _NBUF, body, 0)
    plsc.subcore_barrier()

    pltpu.sync_copy(acc.at[pl.ds(s * _RPT, _RPT)],
                    out_hbm.at[c, pl.ds(s * _RPT, _RPT)])


_msg_kernel = pl.kernel(
    _msg_body,
    out_type=jax.ShapeDtypeStruct((_NC, _NP, _H), jnp.float32),
    mesh=_mesh(),
    scratch_types=[
        pltpu.VMEM((_QR, _IW), jnp.int32),
        pltpu.VMEM((_QR, _IW), jnp.int32),
        pltpu.VMEM((_NBUF * _IW, _D), jnp.float32),
        pltpu.VMEM_SHARED((_NP, _H), jnp.float32),
    ] + [pltpu.SemaphoreType.DMA] * _NBUF,
)


def _tcA0_body(x_ref, w_ref, xw_ref):
    xw_ref[...] = jnp.dot(x_ref[...], w_ref[...],
                          preferred_element_type=jnp.float32)


def _tcA_body(xw_ref, da_ref, db_ref, xs_ref, dv_ref):
    deg = da_ref[:, 0:1] + db_ref[:, 0:1] + 1.0
    dinv = lax.rsqrt(jnp.maximum(deg, 1e-12))
    xs_ref[...] = xw_ref[...] * dinv
    dv_ref[...] = jnp.broadcast_to(dinv, (_BLK, _H))


def _ln_relu(agg, g_ref, be_ref):
    mu = jnp.mean(agg, axis=-1, keepdims=True)
    var = jnp.mean((agg - mu) ** 2, axis=-1, keepdims=True)
    ln = (agg - mu) * lax.rsqrt(var + 1e-5) * g_ref[...] + be_ref[...]
    return jnp.maximum(ln, 0.0)


def _tcB_body(pa_ref, pb_ref, xs_ref, dv_ref, b_ref, g_ref, be_ref, w2_ref,
              out_ref):
    agg = dv_ref[...] * (pa_ref[...] + pb_ref[...] + xs_ref[...]) + b_ref[...]
    h = _ln_relu(agg, g_ref, be_ref)
    xw = jnp.dot(h, w2_ref[...], preferred_element_type=jnp.float32)
    out_ref[...] = xw * dv_ref[:, 0:1]


def _tcC_body(pa_ref, pb_ref, xs_ref, dv_ref, b_ref, g_ref, be_ref,
              wh1_ref, bh1_ref, wh2_ref, bh2_ref, out_ref):
    agg = dv_ref[...] * (pa_ref[...] + pb_ref[...] + xs_ref[...]) + b_ref[...]
    h = _ln_relu(agg, g_ref, be_ref)
    h = jnp.maximum(
        jnp.dot(h, wh1_ref[...], preferred_element_type=jnp.float32)
        + bh1_ref[...], 0.0)
    out_ref[...] = (jnp.dot(h, wh2_ref[...], preferred_element_type=jnp.float32)
                    + bh2_ref[...])


def _row_spec():
    return pl.BlockSpec((_BLK, _H), lambda i: (i, 0))


def _full_spec(r, c):
    return pl.BlockSpec((r, c), lambda i: (0, 0))


_tcA0 = pl.pallas_call(
    _tcA0_body,
    grid=(_GRID,),
    in_specs=[_row_spec(), _full_spec(_D, _H)],
    out_specs=_row_spec(),
    out_shape=jax.ShapeDtypeStruct((_NP, _H), jnp.float32),
)

_tcA = pl.pallas_call(
    _tcA_body,
    grid=(_GRID,),
    in_specs=[_row_spec(), _row_spec(), _row_spec()],
    out_specs=[_row_spec(), _row_spec()],
    out_shape=[jax.ShapeDtypeStruct((_NP, _H), jnp.float32),
               jax.ShapeDtypeStruct((_NP, _H), jnp.float32)],
)

_tcB = pl.pallas_call(
    _tcB_body,
    grid=(_GRID,),
    in_specs=[_row_spec(), _row_spec(), _row_spec(), _row_spec(),
              _full_spec(1, _H), _full_spec(1, _H), _full_spec(1, _H),
              _full_spec(_H, _H)],
    out_specs=_row_spec(),
    out_shape=jax.ShapeDtypeStruct((_NP, _H), jnp.float32),
)

_tcC = pl.pallas_call(
    _tcC_body,
    grid=(_GRID,),
    in_specs=[_row_spec(), _row_spec(), _row_spec(), _row_spec(),
              _full_spec(1, _H), _full_spec(1, _H), _full_spec(1, _H),
              _full_spec(_H, _H), _full_spec(1, _H), _full_spec(_H, _H),
              _full_spec(1, _H)],
    out_specs=_row_spec(),
    out_shape=jax.ShapeDtypeStruct((_NP, _H), jnp.float32),
)


def kernel(x, edge_index, W1, b1, g1, be1, W2, b2, g2, be2, Wh1, bh1, Wh2, bh2):
    f32 = jnp.float32
    x_pad = jnp.pad(x.astype(f32), ((0, _NP - _N), (0, 0)))
    pad = jnp.full((_EP - _E,), _N, jnp.int32)
    src = jnp.concatenate([edge_index[0].astype(jnp.int32), pad]
                          ).reshape(_NR, _IW)
    dst = jnp.concatenate([edge_index[1].astype(jnp.int32), pad]
                          ).reshape(_NR, _IW)

    xw1 = _tcA0(x_pad, W1)
    deg2 = _deg_kernel(dst)
    xs1, dinvb = _tcA(xw1, deg2[0], deg2[1])

    p1 = _msg_kernel(xs1, src, dst)
    xs2 = _tcB(p1[0], p1[1], xs1, dinvb,
               b1.reshape(1, -1), g1.reshape(1, -1), be1.reshape(1, -1), W2)

    p2 = _msg_kernel(xs2, src, dst)
    wh2p = jnp.pad(Wh2, ((0, 0), (0, _H - _C)))
    bh2p = jnp.pad(bh2, (0, _H - _C)).reshape(1, -1)
    outp = _tcC(p2[0], p2[1], xs2, dinvb,
                b2.reshape(1, -1), g2.reshape(1, -1), be2.reshape(1, -1),
                Wh1, bh1.reshape(1, -1), wh2p, bh2p)
    return outp[:_N, :_C]
